# baseline (speedup 1.0000x reference)
import functools

import jax
import jax.numpy as jnp
from jax import lax
from jax.experimental import pallas as pl
from jax.experimental.pallas import tpu as pltpu
from jax.experimental.pallas import tpu_sc as plsc

VOCAB = 1000000
EMBED_DIM = 64
BATCH = 4096
HIST = 200

NC = 2
NS = 16
NW = NC * NS

BB = BATCH // 128
H = 4
N_CHUNKS = HIST // H

_mesh = plsc.VectorSubcoreMesh(
    core_axis_name="c", subcore_axis_name="s", num_cores=NC, num_subcores=NS
)


@functools.partial(
    pl.kernel,
    out_type=jax.ShapeDtypeStruct((HIST, 8, BB, 8, 128), jnp.float32),
    mesh=_mesh,
    scratch_types=[
        pltpu.VMEM((2, H, 128), jnp.int32),
        pltpu.VMEM((2, H, 128, EMBED_DIM), jnp.float32),
        pltpu.VMEM((H, 8, 8, 128), jnp.float32),
        pltpu.SemaphoreType.DMA,
        pltpu.SemaphoreType.DMA,
        pltpu.SemaphoreType.DMA,
        pltpu.SemaphoreType.DMA,
        pltpu.SemaphoreType.DMA,
    ],
    compiler_params=pltpu.CompilerParams(
        use_tc_tiling_on_sc=False, needs_layout_passes=False
    ),
)
def _gather_kernel(
    table_hbm, wordt_hbm, out_hbm, idx_v, rows_v, stage_v,
    sem_i0, sem_i1, sem_g0, sem_g1, sem_o,
):
    wid = lax.axis_index("s") * NC + lax.axis_index("c")
    b0 = wid * 128

    def idx_load(i, slot, sem):
        pltpu.async_copy(
            wordt_hbm.at[pl.ds(i * H, H), pl.ds(b0, 128)], idx_v.at[slot], sem
        )

    def idx_wait(i, slot, sem):
        pltpu.make_async_copy(
            wordt_hbm.at[pl.ds(i * H, H), pl.ds(b0, 128)], idx_v.at[slot], sem
        ).wait()

    def gathers(slot, sem):
        for h in range(H):
            pltpu.async_copy(
                table_hbm.at[idx_v.at[slot, h]], rows_v.at[slot, h], sem
            )

    def gathers_wait(slot, sem):
        for h in range(H):
            pltpu.make_async_copy(
                table_hbm.at[idx_v.at[slot, h]], rows_v.at[slot, h], sem
            ).wait()

    def out_store(i, sem):
        pltpu.async_copy(stage_v, out_hbm.at[pl.ds(i * H, H), :, wid], sem)

    def out_wait(i, sem):
        pltpu.make_async_copy(
            stage_v, out_hbm.at[pl.ds(i * H, H), :, wid], sem
        ).wait()

    iota = lax.iota(jnp.int32, 16)
    zeros = jnp.zeros((16,), jnp.int32)

    def transpose_rows(slot):
        slot_vec = zeros + slot
        for h in range(H):
            h_vec = zeros + h

            def fb_body(fb, carry):
                for fr in range(8):
                    f_vec = zeros + fb * 8 + fr
                    for q in range(8):
                        b_vec = iota + q * 16
                        vals = plsc.load_gather(
                            rows_v, [slot_vec, h_vec, b_vec, f_vec]
                        )
                        stage_v[h, fb, fr, pl.ds(q * 16, 16)] = vals
                return carry

            lax.fori_loop(0, 8, fb_body, 0)

    sems_i = (sem_i0, sem_i1)
    sems_g = (sem_g0, sem_g1)

    idx_load(0, 0, sems_i[0])
    idx_load(1, 1, sems_i[1])
    idx_wait(0, 0, sems_i[0])
    gathers(0, sems_g[0])

    def slot_body(s, i):
        gathers_wait(s, sems_g[s])

        @pl.when(i + 2 < N_CHUNKS)
        def _():
            idx_load(i + 2, s, sems_i[s])

        @pl.when(i + 1 < N_CHUNKS)
        def _():
            o = 1 - s
            idx_wait(i + 1, o, sems_i[o])
            gathers(o, sems_g[o])

        @pl.when(i >= 1)
        def _():
            out_wait(i - 1, sem_o)

        transpose_rows(s)
        out_store(i, sem_o)

    def body(ii, carry):
        i = ii * 2
        slot_body(0, i)
        slot_body(1, i + 1)
        return carry

    lax.fori_loop(0, N_CHUNKS // 2, body, 0)
    out_wait(N_CHUNKS - 1, sem_o)


def kernel(word, table):
    wordt = jnp.swapaxes(word, 0, 1).astype(jnp.int32)
    p = _gather_kernel(table, wordt)
    return p.transpose(2, 4, 0, 1, 3).reshape(BATCH, HIST, EMBED_DIM)


# speedup vs baseline: 1.1518x; 1.1518x over previous
import functools

import jax
import jax.numpy as jnp
from jax import lax
from jax.experimental import pallas as pl
from jax.experimental.pallas import tpu as pltpu
from jax.experimental.pallas import tpu_sc as plsc

VOCAB = 1000000
EMBED_DIM = 64
BATCH = 4096
HIST = 200

NC = 2
NS = 16
NW = NC * NS

BB = BATCH // 128
H = 4
N_CHUNKS = HIST // H

_mesh = plsc.VectorSubcoreMesh(
    core_axis_name="c", subcore_axis_name="s", num_cores=NC, num_subcores=NS
)


@functools.partial(
    pl.kernel,
    out_type=jax.ShapeDtypeStruct((HIST, 8, BB, 8, 128), jnp.float32),
    mesh=_mesh,
    scratch_types=[
        pltpu.VMEM((2, H, 128), jnp.int32),
        pltpu.VMEM((2, H, 128, EMBED_DIM), jnp.float32),
        pltpu.VMEM((H, 8, 8, 128), jnp.float32),
        pltpu.SemaphoreType.DMA,
        pltpu.SemaphoreType.DMA,
        pltpu.SemaphoreType.DMA,
        pltpu.SemaphoreType.DMA,
        pltpu.SemaphoreType.DMA,
    ],
    compiler_params=pltpu.CompilerParams(
        use_tc_tiling_on_sc=False, needs_layout_passes=False
    ),
)
def _gather_kernel(
    table_hbm, wordt_hbm, out_hbm, idx_v, rows_v, stage_v,
    sem_i0, sem_i1, sem_g0, sem_g1, sem_o,
):
    wid = lax.axis_index("s") * NC + lax.axis_index("c")
    b0 = wid * 128

    def idx_load(i, slot, sem):
        pltpu.async_copy(
            wordt_hbm.at[pl.ds(i * H, H), pl.ds(b0, 128)], idx_v.at[slot], sem
        )

    def idx_wait(i, slot, sem):
        pltpu.make_async_copy(
            wordt_hbm.at[pl.ds(i * H, H), pl.ds(b0, 128)], idx_v.at[slot], sem
        ).wait()

    def gathers(slot, sem):
        for h in range(H):
            pltpu.async_copy(
                table_hbm.at[idx_v.at[slot, h]], rows_v.at[slot, h], sem
            )

    def gathers_wait(slot, sem):
        for h in range(H):
            pltpu.make_async_copy(
                table_hbm.at[idx_v.at[slot, h]], rows_v.at[slot, h], sem
            ).wait()

    def out_store(i, sem):
        pltpu.async_copy(stage_v, out_hbm.at[pl.ds(i * H, H), :, wid], sem)

    def out_wait(i, sem):
        pltpu.make_async_copy(
            stage_v, out_hbm.at[pl.ds(i * H, H), :, wid], sem
        ).wait()

    iota = lax.iota(jnp.int32, 16)
    zeros = jnp.zeros((16,), jnp.int32)

    def transpose_rows(slot):
        slot_vec = zeros + slot
        for h in range(H):
            h_vec = zeros + h

            def fb_body(fb, carry):
                for fr in range(8):
                    f_vec = zeros + fb * 8 + fr
                    vals = [
                        plsc.load_gather(
                            rows_v, [slot_vec, h_vec, iota + q * 16, f_vec]
                        )
                        for q in range(8)
                    ]
                    for q in range(8):
                        stage_v[h, fb, fr, pl.ds(q * 16, 16)] = vals[q]
                return carry

            lax.fori_loop(0, 8, fb_body, 0)

    sems_i = (sem_i0, sem_i1)
    sems_g = (sem_g0, sem_g1)

    idx_load(0, 0, sems_i[0])
    idx_load(1, 1, sems_i[1])
    idx_wait(0, 0, sems_i[0])
    gathers(0, sems_g[0])

    def slot_body(s, i):
        gathers_wait(s, sems_g[s])

        @pl.when(i + 2 < N_CHUNKS)
        def _():
            idx_load(i + 2, s, sems_i[s])

        @pl.when(i + 1 < N_CHUNKS)
        def _():
            o = 1 - s
            idx_wait(i + 1, o, sems_i[o])
            gathers(o, sems_g[o])

        @pl.when(i >= 1)
        def _():
            out_wait(i - 1, sem_o)

        transpose_rows(s)
        out_store(i, sem_o)

    def body(ii, carry):
        i = ii * 2
        slot_body(0, i)
        slot_body(1, i + 1)
        return carry

    lax.fori_loop(0, N_CHUNKS // 2, body, 0)
    out_wait(N_CHUNKS - 1, sem_o)


def kernel(word, table):
    wordt = jnp.swapaxes(word, 0, 1).astype(jnp.int32)
    p = _gather_kernel(table, wordt)
    return p.transpose(2, 4, 0, 1, 3).reshape(BATCH, HIST, EMBED_DIM)


# speedup vs baseline: 2.0429x; 1.7737x over previous
import functools

import jax
import jax.numpy as jnp
from jax import lax
from jax.experimental import pallas as pl
from jax.experimental.pallas import tpu as pltpu
from jax.experimental.pallas import tpu_sc as plsc

VOCAB = 1000000
EMBED_DIM = 64
BATCH = 4096
HIST = 200

NC = 2
NS = 16
NW = NC * NS

B = BATCH * HIST
B_PER_W = B // NW
C = 512
G = 512
K = C // G
N_CHUNKS = B_PER_W // C

_mesh = plsc.VectorSubcoreMesh(
    core_axis_name="c", subcore_axis_name="s", num_cores=NC, num_subcores=NS
)


@functools.partial(
    pl.kernel,
    out_type=jax.ShapeDtypeStruct((B, 2 * EMBED_DIM), jnp.float32),
    mesh=_mesh,
    scratch_types=[
        pltpu.VMEM((2, K, G), jnp.int32),
        pltpu.VMEM((2, C, EMBED_DIM), jnp.float32),
        pltpu.SemaphoreType.DMA,
        pltpu.SemaphoreType.DMA,
        pltpu.SemaphoreType.DMA,
        pltpu.SemaphoreType.DMA,
        pltpu.SemaphoreType.DMA,
        pltpu.SemaphoreType.DMA,
    ],
    compiler_params=pltpu.CompilerParams(use_tc_tiling_on_sc=False),
)
def _gather_kernel(
    table_hbm, word_hbm, out_hbm, idx_v, rows_v,
    sem_i0, sem_i1, sem_g0, sem_g1, sem_o0, sem_o1,
):
    wid = lax.axis_index("s") * NC + lax.axis_index("c")
    row0 = wid * (B_PER_W // G)

    def idx_load(i, slot, sem):
        pltpu.async_copy(
            word_hbm.at[pl.ds(row0 + i * K, K), :], idx_v.at[slot], sem
        )

    def idx_wait(i, slot, sem):
        pltpu.make_async_copy(
            word_hbm.at[pl.ds(row0 + i * K, K), :], idx_v.at[slot], sem
        ).wait()

    def gathers(slot, sem):
        for j in range(K):
            pltpu.async_copy(
                table_hbm.at[idx_v.at[slot, j]],
                rows_v.at[slot, pl.ds(j * G, G), :],
                sem,
            )

    def gathers_wait(slot, sem):
        for j in range(K):
            pltpu.make_async_copy(
                table_hbm.at[idx_v.at[slot, j]],
                rows_v.at[slot, pl.ds(j * G, G), :],
                sem,
            ).wait()

    def out_store(i, slot, sem):
        base = wid * B_PER_W + i * C
        pltpu.async_copy(
            rows_v.at[slot], out_hbm.at[pl.ds(base, C), pl.ds(0, EMBED_DIM)], sem
        )

    def out_wait(i, slot, sem):
        base = wid * B_PER_W + i * C
        pltpu.make_async_copy(
            rows_v.at[slot], out_hbm.at[pl.ds(base, C), pl.ds(0, EMBED_DIM)], sem
        ).wait()

    sems_i = (sem_i0, sem_i1)
    sems_g = (sem_g0, sem_g1)
    sems_o = (sem_o0, sem_o1)

    idx_load(0, 0, sems_i[0])
    idx_load(1, 1, sems_i[1])
    idx_wait(0, 0, sems_i[0])
    gathers(0, sems_g[0])

    def slot_body(s, i):
        gathers_wait(s, sems_g[s])
        out_store(i, s, sems_o[s])

        @pl.when(i + 2 < N_CHUNKS)
        def _():
            idx_load(i + 2, s, sems_i[s])

        @pl.when(i + 1 < N_CHUNKS)
        def _():
            o = 1 - s
            idx_wait(i + 1, o, sems_i[o])

            @pl.when(i >= 1)
            def _():
                out_wait(i - 1, o, sems_o[o])

            gathers(o, sems_g[o])

    def body(ii, carry):
        i = ii * 2
        slot_body(0, i)
        slot_body(1, i + 1)
        return carry

    lax.fori_loop(0, N_CHUNKS // 2, body, 0)
    out_wait(N_CHUNKS - 2, 0, sems_o[0])
    out_wait(N_CHUNKS - 1, 1, sems_o[1])


def kernel(word, table):
    word_flat = word.reshape(-1).astype(jnp.int32).reshape(B // G, G)
    out = _gather_kernel(table, word_flat)
    return out.reshape(BATCH, HIST, 2 * EMBED_DIM)[..., :EMBED_DIM]
